# Initial kernel scaffold; baseline (speedup 1.0000x reference)
import jax
import jax.numpy as jnp
from jax.experimental import pallas as pl


def kernel(ground_atoms, clause_weight):
    raise NotImplementedError("write your pallas kernel here")


# speedup vs baseline: 1.6731x; 1.6731x over previous
import functools

import jax
import jax.numpy as jnp
import numpy as np
from jax.experimental import pallas as pl
from jax.experimental.pallas import tpu as pltpu

_GATHER_IDX = (3, 17, 42, 63, 77, 99, 110, 120)
_SIGNS = (-1.0, 1.0, -1.0, 1.0, -1.0, 1.0, -1.0, 1.0)
_D = 128
_K = 8

def _body(w_ref, x_ref, out_ref, delta_ref):
    x = x_ref[...]
    lane = jax.lax.broadcasted_iota(jnp.int32, (1, _D), 1)
    s_full = jnp.zeros((1, _D), jnp.float32)
    for c, s in zip(_GATHER_IDX, _SIGNS):
        s_full = jnp.where(lane == c, jnp.float32(s), s_full)
    mask = s_full * s_full
    e = jnp.exp(x * s_full) * mask
    ssum = jnp.sum(e, axis=-1, keepdims=True)
    scale = w_ref[0, 0] / ssum
    scattered = e * (s_full * scale)
    out_ref[...] = scattered
    delta_ref[...] = jnp.concatenate(
        [scattered[:, c:c + 1] for c in _GATHER_IDX], axis=1)


@functools.partial(jax.jit, static_argnames=())
def kernel(ground_atoms, clause_weight):
    n, d = ground_atoms.shape
    rows = 1024
    grid = (n // rows,)
    w = jnp.reshape(clause_weight.astype(jnp.float32), (1, 1))
    scattered, delta = pl.pallas_call(
        _body,
        grid=grid,
        in_specs=[
            pl.BlockSpec(memory_space=pltpu.SMEM),
            pl.BlockSpec((rows, d), lambda i: (i, 0)),
        ],
        out_specs=[
            pl.BlockSpec((rows, d), lambda i: (i, 0)),
            pl.BlockSpec((rows, _K), lambda i: (i, 0)),
        ],
        out_shape=[
            jax.ShapeDtypeStruct((n, d), jnp.float32),
            jax.ShapeDtypeStruct((n, _K), jnp.float32),
        ],
    )(w, ground_atoms)
    return (scattered, delta)
